# baseline (speedup 1.0000x reference)
import functools
import math

import jax
import jax.numpy as jnp
from jax import lax
from jax.experimental import pallas as pl
from jax.experimental.pallas import tpu as pltpu
from jax.experimental.pallas import tpu_sc as plsc

_BATCH = 4
_MAX_NODES = 2500
_MAX_EDGES = 80000
_P = 1024
_NB = 20
_H = 128
_NI = 3
_STEP = 0.1
_ES = 50
_ESP = 64
_CH = 64
_N = _BATCH * _MAX_NODES
_NP_ = 10240
_E = _BATCH * _MAX_EDGES
_LN2 = math.log(2.0)

_NC = 2
_NS = 16
_NW = _NC * _NS
_C = 80
_ECH = _E // _NW // _C
_RPT = _NP_ // _NS


def _mesh():
    return plsc.VectorSubcoreMesh(
        core_axis_name="c", subcore_axis_name="s",
        num_cores=_NC, num_subcores=_NS)


_COLS_A = [32 * g + j for g in range(_H // 32) for j in range(16)]
_COLS_B = [32 * g + 16 + j for g in range(_H // 32) for j in range(16)]


def _ssp(x):
    return jnp.maximum(x, 0.0) + jnp.log(1.0 + jnp.exp(-jnp.abs(x))) - _LN2


def _sc_gather_call(table, idx, nch):
    total = _NW * nch * _C

    def body(table_hbm, idx_hbm, out_hbm, idx_v, rows_v, sem):
        c = lax.axis_index("c")
        s = lax.axis_index("s")
        wid = s * _NC + c
        pltpu.sync_copy(idx_hbm.at[wid], idx_v)

        def step(j, carry):
            pltpu.async_copy(table_hbm.at[idx_v.at[j]], rows_v, sem).wait()
            pltpu.sync_copy(rows_v, out_hbm.at[pl.ds(wid * nch * _C + j * _C, _C)])
            return carry

        lax.fori_loop(0, nch, step, 0)

    f = pl.kernel(
        body,
        out_type=jax.ShapeDtypeStruct((total, _H), jnp.float32),
        mesh=_mesh(),
        scratch_types=[
            pltpu.VMEM((nch, _C), jnp.int32),
            pltpu.VMEM((_C, _H), jnp.float32),
            pltpu.SemaphoreType.DMA,
        ],
    )
    return f(table, idx)


def _sc_interact_call(h, src, dst, gates, zeros):
    cc_ = 40
    nchk = _E // _NW // cc_
    gw = _H // 2

    def body(h_hbm, src_hbm, dst_hbm, g_hbm, z_hbm, out_hbm,
             src_v, rows_v, gates_v, dst_v,
             lsem0, lsem1, gsem0, gsem1, ssem0, ssem1, agg_sh):
        c = lax.axis_index("c")
        s = lax.axis_index("s")
        wid = s * _NC + c
        lsem = (lsem0, lsem1)
        gsem = (gsem0, gsem1)
        ssem = (ssem0, ssem1)
        pltpu.sync_copy(z_hbm, agg_sh.at[pl.ds(s * _RPT, _RPT)])
        plsc.subcore_barrier()

        ebase = wid * nchk * cc_

        def linear_issue(j, b):
            base = ebase + j * cc_
            pltpu.async_copy(src_hbm.at[pl.ds(base, cc_)], src_v.at[b], lsem[b])
            pltpu.async_copy(dst_hbm.at[pl.ds(base, cc_)], dst_v.at[j % 4], lsem[b])
            pltpu.async_copy(g_hbm.at[pl.ds(base * gw, cc_ * gw)],
                             gates_v.at[pl.ds(b * cc_ * gw, cc_ * gw)], lsem[b])

        def linear_wait(b):
            pltpu.make_async_copy(src_hbm.at[pl.ds(0, cc_)], src_v.at[b], lsem[b]).wait()
            pltpu.make_async_copy(dst_hbm.at[pl.ds(0, cc_)], src_v.at[b], lsem[b]).wait()
            pltpu.make_async_copy(g_hbm.at[pl.ds(0, cc_ * gw)],
                                  gates_v.at[pl.ds(b * cc_ * gw, cc_ * gw)], lsem[b]).wait()

        def gather_issue(b):
            pltpu.async_copy(h_hbm.at[src_v.at[b]], rows_v.at[b], gsem[b])

        def gather_wait(b):
            pltpu.make_async_copy(h_hbm.at[src_v.at[b]], rows_v.at[b], gsem[b]).wait()

        def scatter_issue(j, b):
            pltpu.async_copy(rows_v.at[b], agg_sh.at[dst_v.at[j % 4]], ssem[b], add=True)

        def scatter_wait(b):
            pltpu.make_async_copy(rows_v.at[b], agg_sh.at[pl.ds(0, cc_)], ssem[b]).wait()

        linear_issue(0, 0)
        linear_issue(1, 1)
        linear_wait(0)
        gather_issue(0)

        def gbody(g, carry):
            for b in (0, 1):
                j = 2 * g + b
                nb = 1 - b
                nxt = j + 1 < nchk if b else True

                @pl.when((j >= 1) & nxt)
                def _():
                    scatter_wait(nb)

                @pl.when(nxt)
                def _():
                    linear_wait(nb)
                    gather_issue(nb)

                gather_wait(b)

                @plsc.parallel_loop(0, cc_, 1, unroll=4)
                def mrow(r):
                    for kk in range(_H // 32):
                        goff = pl.multiple_of((b * cc_ + r) * gw + kk * 16, 16)
                        vg = gates_v[pl.ds(goff, 16)]
                        glo = lax.bitcast_convert_type(vg << 16, jnp.float32)
                        ghi = lax.bitcast_convert_type(
                            vg & jnp.int32(-65536), jnp.float32)
                        slo = pl.ds(kk * 32, 16)
                        shi = pl.ds(kk * 32 + 16, 16)
                        rows_v[b, r, slo] = rows_v[b, r, slo] * glo
                        rows_v[b, r, shi] = rows_v[b, r, shi] * ghi

                scatter_issue(j, b)

                @pl.when(j + 2 < nchk)
                def _():
                    linear_issue(j + 2, b)
            return carry

        lax.fori_loop(0, nchk // 2, gbody, 0)
        scatter_wait(0)
        scatter_wait(1)
        plsc.subcore_barrier()
        for k in range(_RPT // cc_):
            off = s * _RPT + k * cc_
            pltpu.sync_copy(agg_sh.at[pl.ds(off, cc_)], rows_v.at[0])
            pltpu.sync_copy(rows_v.at[0], out_hbm.at[c, pl.ds(off, cc_)])

    f = pl.kernel(
        body,
        out_type=jax.ShapeDtypeStruct((2, _NP_, _H), jnp.float32),
        mesh=_mesh(),
        scratch_types=[
            pltpu.VMEM((2, cc_), jnp.int32),
            pltpu.VMEM((2, cc_, _H), jnp.float32),
            pltpu.VMEM((2 * cc_ * _H // 2,), jnp.int32),
            pltpu.VMEM((4, cc_), jnp.int32),
            pltpu.SemaphoreType.DMA,
            pltpu.SemaphoreType.DMA,
            pltpu.SemaphoreType.DMA,
            pltpu.SemaphoreType.DMA,
            pltpu.SemaphoreType.DMA,
            pltpu.SemaphoreType.DMA,
            pltpu.VMEM_SHARED((_NP_, _H), jnp.float32),
        ],
    )
    return f(h, src, dst, gates, zeros)


def _gates_call(ef2d, w1p, b1, w2ab, b2ab):
    be = 512

    def to_bf16_bits(x):
        u = lax.bitcast_convert_type(x, jnp.uint32)
        return (u + 0x7FFF + ((u >> 16) & 1)) >> 16

    def body(ef_ref, w1_ref, b1_ref, w2_ref, b2_ref, g_ref):
        ef = ef_ref[...]
        ki = lax.broadcasted_iota(jnp.int32, (1, _ESP), 1)
        mu = ki.astype(jnp.float32) * _STEP
        msk = ki < _ES
        es = jnp.exp(-((ef - mu) ** 2) * (1.0 / (2.0 * _STEP * _STEP)))
        es = jnp.where(msk, es, 0.0)
        x = jnp.dot(es, w1_ref[...], preferred_element_type=jnp.float32)
        x = _ssp(x + b1_ref[...])
        g = jnp.dot(x, w2_ref[...], preferred_element_type=jnp.float32) + b2_ref[...]
        w = to_bf16_bits(g[:, : _H // 2]) | (to_bf16_bits(g[:, _H // 2:]) << 16)
        g_ref[...] = lax.bitcast_convert_type(w, jnp.int32)

    return pl.pallas_call(
        body,
        grid=(_E // be,),
        in_specs=[
            pl.BlockSpec((be, 1), lambda i: (i, 0)),
            pl.BlockSpec((_ESP, _H), lambda i: (0, 0)),
            pl.BlockSpec((1, _H), lambda i: (0, 0)),
            pl.BlockSpec((_H, _H), lambda i: (0, 0)),
            pl.BlockSpec((1, _H), lambda i: (0, 0)),
        ],
        out_specs=pl.BlockSpec((be, _H // 2), lambda i: (i, 0)),
        out_shape=jax.ShapeDtypeStruct((_E, _H // 2), jnp.int32),
        interpret=False,
    )(ef2d, w1p, b1, w2ab, b2ab)


def _state_call(p, h, w1, b1, w2, b2):
    bn = 512

    def body(p0_ref, p1_ref, h_ref, w1_ref, b1_ref, w2_ref, b2_ref, out_ref):
        agg = p0_ref[...] + p1_ref[...]
        x = _ssp(jnp.dot(agg, w1_ref[...], preferred_element_type=jnp.float32) + b1_ref[...])
        out_ref[...] = h_ref[...] + jnp.dot(x, w2_ref[...], preferred_element_type=jnp.float32) + b2_ref[...]

    return pl.pallas_call(
        body,
        grid=(_NP_ // bn,),
        in_specs=[
            pl.BlockSpec((bn, _H), lambda i: (i, 0)),
            pl.BlockSpec((bn, _H), lambda i: (i, 0)),
            pl.BlockSpec((bn, _H), lambda i: (i, 0)),
            pl.BlockSpec((_H, _H), lambda i: (0, 0)),
            pl.BlockSpec((1, _H), lambda i: (0, 0)),
            pl.BlockSpec((_H, _H), lambda i: (0, 0)),
            pl.BlockSpec((1, _H), lambda i: (0, 0)),
        ],
        out_specs=pl.BlockSpec((bn, _H), lambda i: (i, 0)),
        out_shape=jax.ShapeDtypeStruct((_NP_, _H), jnp.float32),
        interpret=False,
    )(p[0], p[1], h, w1, b1, w2, b2)


def _readout_call(nbr, crd, ab, wh, wc_c, bc2, wrh, wrab, brp):
    bp = 128
    rr_ = bp * _NB

    def body(nbr_ref, crd_ref, ab_ref, wh_ref, wc_ref, bc_ref,
             wrh_ref, wrab_ref, br_ref, out_ref):
        x = jnp.dot(nbr_ref[...], wh_ref[...], preferred_element_type=jnp.float32)
        x = x + jnp.dot(crd_ref[...], wc_ref[...], preferred_element_type=jnp.float32)
        ch = _ssp(x + bc_ref[...])
        rcol = lax.broadcasted_iota(jnp.int32, (bp, rr_), 1)
        prow = lax.broadcasted_iota(jnp.int32, (bp, rr_), 0) * _NB
        seg = ((rcol >= prow) & (rcol < prow + _NB)).astype(jnp.float32)
        csum = jnp.dot(seg, ch, preferred_element_type=jnp.float32)
        out_ref[...] = (jnp.dot(csum, wrh_ref[...], preferred_element_type=jnp.float32)
                        + jnp.dot(ab_ref[...], wrab_ref[...], preferred_element_type=jnp.float32)
                        + br_ref[...])

    return pl.pallas_call(
        body,
        grid=(_P // bp,),
        in_specs=[
            pl.BlockSpec((rr_, _H), lambda i: (i, 0)),
            pl.BlockSpec((rr_, 8), lambda i: (i, 0)),
            pl.BlockSpec((bp, 8), lambda i: (i, 0)),
            pl.BlockSpec((_H, _CH), lambda i: (0, 0)),
            pl.BlockSpec((8, _CH), lambda i: (0, 0)),
            pl.BlockSpec((1, _CH), lambda i: (0, 0)),
            pl.BlockSpec((_CH, _H), lambda i: (0, 0)),
            pl.BlockSpec((8, _H), lambda i: (0, 0)),
            pl.BlockSpec((1, _H), lambda i: (0, 0)),
        ],
        out_specs=pl.BlockSpec((bp, _H), lambda i: (i, 0)),
        out_shape=jax.ShapeDtypeStruct((_P, _H), jnp.float32),
        interpret=False,
    )(nbr, crd, ab, wh, wc_c, bc2, wrh, wrab, brp)


def kernel(nodes, num_nodes, edges, edges_features, num_edges, node_id_neighbors,
           internal_coordinates_neighbors, num_neighbors, A_dist, B_dist, atom_emb,
           msg_W1, msg_b1, msg_W2, msg_b2, st_W1, st_b1, st_W2, st_b2, Wc, bc, Wr, br):
    f32 = jnp.float32
    n = nodes.reshape(_N)
    off = (jnp.arange(_BATCH, dtype=edges.dtype) * _MAX_NODES)[:, None, None]
    e = (edges + off).reshape(_E, 2)
    src = e[:, 0]
    dst = e[:, 1]
    ef2d = edges_features.reshape(_E, 1).astype(f32)

    w1p = jnp.zeros((_NI, _ESP, _H), f32).at[:, :_ES, :].set(msg_W1)

    npad = jnp.concatenate([n, jnp.zeros((_NP_ - _N,), n.dtype)]).reshape(_NW, 4, _C)
    h = _sc_gather_call(atom_emb.astype(f32), npad, 4)

    zeros = jnp.zeros((_RPT, _H), f32)
    cab = jnp.asarray(_COLS_A + _COLS_B, dtype=jnp.int32)
    w2ab, b2ab = msg_W2[:, :, cab], msg_b2[:, cab]
    gates = [_gates_call(ef2d, w1p[i], msg_b1[i].reshape(1, _H),
                         w2ab[i], b2ab[i].reshape(1, _H))
             for i in range(_NI)]

    for i in range(_NI):
        p = _sc_interact_call(h, src, dst, gates[i].reshape(_E * (_H // 2)), zeros)
        h = _state_call(p, h, st_W1[i], st_b1[i].reshape(1, _H),
                        st_W2[i], st_b2[i].reshape(1, _H))

    nch_nbr = _P * _NB // _NW // _C
    nidx = node_id_neighbors.reshape(_NW, nch_nbr, _C)
    nbr = _sc_gather_call(h, nidx, nch_nbr)
    crd = jnp.concatenate(
        [internal_coordinates_neighbors.reshape(_P * _NB, 3).astype(f32),
         jnp.zeros((_P * _NB, 5), f32)], axis=1)
    ab = jnp.concatenate(
        [A_dist[:, None].astype(f32), B_dist[:, None].astype(f32),
         jnp.zeros((_P, 6), f32)], axis=1)
    wh = Wc[:_H].astype(f32)
    wc_c = jnp.zeros((8, _CH), f32).at[:3].set(Wc[_H:_H + 3])
    wrh = jnp.zeros((_CH, _H), f32).at[:, :6].set(Wr[:_CH])
    wrab = jnp.zeros((8, _H), f32).at[:2, :6].set(Wr[_CH:_CH + 2])
    brp = jnp.zeros((1, _H), f32).at[0, :6].set(br)
    out = _readout_call(nbr, crd, ab, wh, wc_c, bc.reshape(1, _CH), wrh, wrab, brp)
    return out[:, :6]


# speedup vs baseline: 1.0919x; 1.0919x over previous
import functools
import math

import jax
import jax.numpy as jnp
from jax import lax
from jax.experimental import pallas as pl
from jax.experimental.pallas import tpu as pltpu
from jax.experimental.pallas import tpu_sc as plsc

_BATCH = 4
_MAX_NODES = 2500
_MAX_EDGES = 80000
_P = 1024
_NB = 20
_H = 128
_NI = 3
_STEP = 0.1
_ES = 50
_ESP = 64
_CH = 64
_N = _BATCH * _MAX_NODES
_NP_ = 10240
_E = _BATCH * _MAX_EDGES
_LN2 = math.log(2.0)

_NC = 2
_NS = 16
_NW = _NC * _NS
_C = 80
_ECH = _E // _NW // _C
_RPT = _NP_ // _NS


def _mesh():
    return plsc.VectorSubcoreMesh(
        core_axis_name="c", subcore_axis_name="s",
        num_cores=_NC, num_subcores=_NS)


_COLS_A = [32 * g + j for g in range(_H // 32) for j in range(16)]
_COLS_B = [32 * g + 16 + j for g in range(_H // 32) for j in range(16)]


def _ssp(x):
    return jnp.maximum(x, 0.0) + jnp.log(1.0 + jnp.exp(-jnp.abs(x))) - _LN2


def _sc_gather_call(table, idx, nch):
    total = _NW * nch * _C

    def body(table_hbm, idx_hbm, out_hbm, idx_v, rows_v, sem):
        c = lax.axis_index("c")
        s = lax.axis_index("s")
        wid = s * _NC + c
        pltpu.sync_copy(idx_hbm.at[wid], idx_v)

        def step(j, carry):
            pltpu.async_copy(table_hbm.at[idx_v.at[j]], rows_v, sem).wait()
            pltpu.sync_copy(rows_v, out_hbm.at[pl.ds(wid * nch * _C + j * _C, _C)])
            return carry

        lax.fori_loop(0, nch, step, 0)

    f = pl.kernel(
        body,
        out_type=jax.ShapeDtypeStruct((total, _H), jnp.float32),
        mesh=_mesh(),
        scratch_types=[
            pltpu.VMEM((nch, _C), jnp.int32),
            pltpu.VMEM((_C, _H), jnp.float32),
            pltpu.SemaphoreType.DMA,
        ],
    )
    return f(table, idx)


def _sc_interact_call(h, src, dst, gates, zeros):
    cc_ = 40
    nchk = _E // _NW // cc_
    gw = _H // 2

    def body(h_hbm, src_hbm, dst_hbm, g_hbm, z_hbm, out_hbm,
             src_v, rows_v, gates_v, dst_v,
             lsem0, lsem1, gsem0, gsem1, ssem0, ssem1, agg_sh):
        c = lax.axis_index("c")
        s = lax.axis_index("s")
        wid = s * _NC + c
        lsem = (lsem0, lsem1)
        gsem = (gsem0, gsem1)
        ssem = (ssem0, ssem1)
        pltpu.sync_copy(z_hbm, agg_sh.at[pl.ds(s * _RPT, _RPT)])
        plsc.subcore_barrier()

        ebase = wid * nchk * cc_

        def linear_issue(j, b):
            base = ebase + j * cc_
            pltpu.async_copy(src_hbm.at[pl.ds(base, cc_)], src_v.at[b], lsem[b])
            pltpu.async_copy(dst_hbm.at[pl.ds(base, cc_)], dst_v.at[j % 4], lsem[b])
            pltpu.async_copy(g_hbm.at[pl.ds(base * gw, cc_ * gw)],
                             gates_v.at[pl.ds(b * cc_ * gw, cc_ * gw)], lsem[b])

        def linear_wait(b):
            pltpu.make_async_copy(src_hbm.at[pl.ds(0, cc_)], src_v.at[b], lsem[b]).wait()
            pltpu.make_async_copy(dst_hbm.at[pl.ds(0, cc_)], src_v.at[b], lsem[b]).wait()
            pltpu.make_async_copy(g_hbm.at[pl.ds(0, cc_ * gw)],
                                  gates_v.at[pl.ds(b * cc_ * gw, cc_ * gw)], lsem[b]).wait()

        def gather_issue(b):
            pltpu.async_copy(h_hbm.at[src_v.at[b]], rows_v.at[b], gsem[b])

        def gather_wait(b):
            pltpu.make_async_copy(h_hbm.at[src_v.at[b]], rows_v.at[b], gsem[b]).wait()

        def scatter_issue(j, b):
            pltpu.async_copy(rows_v.at[b], agg_sh.at[dst_v.at[j % 4]], ssem[b], add=True)

        def scatter_wait(b):
            pltpu.make_async_copy(rows_v.at[b], agg_sh.at[pl.ds(0, cc_)], ssem[b]).wait()

        linear_issue(0, 0)
        linear_issue(1, 1)
        linear_wait(0)
        gather_issue(0)

        def gbody(g, carry):
            for b in (0, 1):
                j = 2 * g + b
                nb = 1 - b
                nxt = j + 1 < nchk if b else True

                @pl.when((j >= 1) & nxt)
                def _():
                    scatter_wait(nb)

                @pl.when(nxt)
                def _():
                    linear_wait(nb)
                    gather_issue(nb)

                gather_wait(b)

                @plsc.parallel_loop(0, cc_, 1, unroll=4)
                def mrow(r):
                    for kk in range(_H // 32):
                        goff = pl.multiple_of((b * cc_ + r) * gw + kk * 16, 16)
                        vg = gates_v[pl.ds(goff, 16)]
                        glo = lax.bitcast_convert_type(vg << 16, jnp.float32)
                        ghi = lax.bitcast_convert_type(
                            vg & jnp.int32(-65536), jnp.float32)
                        slo = pl.ds(kk * 32, 16)
                        shi = pl.ds(kk * 32 + 16, 16)
                        rows_v[b, r, slo] = rows_v[b, r, slo] * glo
                        rows_v[b, r, shi] = rows_v[b, r, shi] * ghi

                scatter_issue(j, b)

                @pl.when(j + 2 < nchk)
                def _():
                    linear_issue(j + 2, b)
            return carry

        lax.fori_loop(0, nchk // 2, gbody, 0)
        scatter_wait(0)
        scatter_wait(1)
        plsc.subcore_barrier()
        for k in range(_RPT // cc_):
            off = s * _RPT + k * cc_
            pltpu.sync_copy(agg_sh.at[pl.ds(off, cc_)], rows_v.at[0])
            pltpu.sync_copy(rows_v.at[0], out_hbm.at[c, pl.ds(off, cc_)])

    f = pl.kernel(
        body,
        out_type=jax.ShapeDtypeStruct((2, _NP_, _H), jnp.float32),
        mesh=_mesh(),
        scratch_types=[
            pltpu.VMEM((2, cc_), jnp.int32),
            pltpu.VMEM((2, cc_, _H), jnp.float32),
            pltpu.VMEM((2 * cc_ * _H // 2,), jnp.int32),
            pltpu.VMEM((4, cc_), jnp.int32),
            pltpu.SemaphoreType.DMA,
            pltpu.SemaphoreType.DMA,
            pltpu.SemaphoreType.DMA,
            pltpu.SemaphoreType.DMA,
            pltpu.SemaphoreType.DMA,
            pltpu.SemaphoreType.DMA,
            pltpu.VMEM_SHARED((_NP_, _H), jnp.float32),
        ],
    )
    return f(h, src, dst, gates, zeros)


def _gates_call(ef2d, w1p, b1, w2ab, b2ab):
    be = 640

    def to_bf16_bits(x):
        u = lax.bitcast_convert_type(x, jnp.uint32)
        return (u + 0x7FFF + ((u >> 16) & 1)) >> 16

    def body(ef_ref, w1_ref, b1_ref, w2_ref, b2_ref, g_ref):
        ef = ef_ref[...]
        ki = lax.broadcasted_iota(jnp.int32, (1, _ESP), 1)
        mu = ki.astype(jnp.float32) * _STEP
        msk = ki < _ES
        es = jnp.exp(-((ef - mu) ** 2) * (1.0 / (2.0 * _STEP * _STEP)))
        es = jnp.where(msk, es, 0.0)
        x = jnp.dot(es.astype(jnp.bfloat16), w1_ref[...],
                    preferred_element_type=jnp.float32)
        x = _ssp(x + b1_ref[...])
        g = jnp.dot(x.astype(jnp.bfloat16), w2_ref[...],
                    preferred_element_type=jnp.float32) + b2_ref[...]
        w = to_bf16_bits(g[:, : _H // 2]) | (to_bf16_bits(g[:, _H // 2:]) << 16)
        g_ref[...] = lax.bitcast_convert_type(w, jnp.int32)

    return pl.pallas_call(
        body,
        grid=(_E // be,),
        in_specs=[
            pl.BlockSpec((be, 1), lambda i: (i, 0)),
            pl.BlockSpec((_ESP, _H), lambda i: (0, 0)),
            pl.BlockSpec((1, _H), lambda i: (0, 0)),
            pl.BlockSpec((_H, _H), lambda i: (0, 0)),
            pl.BlockSpec((1, _H), lambda i: (0, 0)),
        ],
        out_specs=pl.BlockSpec((be, _H // 2), lambda i: (i, 0)),
        out_shape=jax.ShapeDtypeStruct((_E, _H // 2), jnp.int32),
        interpret=False,
    )(ef2d, w1p, b1, w2ab, b2ab)


def _emb_call(n2d, emb_pad):
    bn = 512

    def body(n_ref, emb_ref, out_ref):
        oh = (n_ref[...] == lax.broadcasted_iota(jnp.int32, (1, _H), 1))
        out_ref[...] = jnp.dot(oh.astype(jnp.float32), emb_ref[...],
                               preferred_element_type=jnp.float32)

    return pl.pallas_call(
        body,
        grid=(_NP_ // bn,),
        in_specs=[
            pl.BlockSpec((bn, 1), lambda i: (i, 0)),
            pl.BlockSpec((_H, _H), lambda i: (0, 0)),
        ],
        out_specs=pl.BlockSpec((bn, _H), lambda i: (i, 0)),
        out_shape=jax.ShapeDtypeStruct((_NP_, _H), jnp.float32),
        interpret=False,
    )(n2d, emb_pad)


def _state_call(p, h, w1, b1, w2, b2):
    bn = 512

    def body(p0_ref, p1_ref, h_ref, w1_ref, b1_ref, w2_ref, b2_ref, out_ref):
        agg = p0_ref[...] + p1_ref[...]
        x = _ssp(jnp.dot(agg, w1_ref[...], preferred_element_type=jnp.float32) + b1_ref[...])
        out_ref[...] = h_ref[...] + jnp.dot(x, w2_ref[...], preferred_element_type=jnp.float32) + b2_ref[...]

    return pl.pallas_call(
        body,
        grid=(_NP_ // bn,),
        in_specs=[
            pl.BlockSpec((bn, _H), lambda i: (i, 0)),
            pl.BlockSpec((bn, _H), lambda i: (i, 0)),
            pl.BlockSpec((bn, _H), lambda i: (i, 0)),
            pl.BlockSpec((_H, _H), lambda i: (0, 0)),
            pl.BlockSpec((1, _H), lambda i: (0, 0)),
            pl.BlockSpec((_H, _H), lambda i: (0, 0)),
            pl.BlockSpec((1, _H), lambda i: (0, 0)),
        ],
        out_specs=pl.BlockSpec((bn, _H), lambda i: (i, 0)),
        out_shape=jax.ShapeDtypeStruct((_NP_, _H), jnp.float32),
        interpret=False,
    )(p[0], p[1], h, w1, b1, w2, b2)


def _readout_call(nbr, crd, ab, wh, wc_c, bc2, wrh, wrab, brp):
    bp = 128
    rr_ = bp * _NB

    def body(nbr_ref, crd_ref, ab_ref, wh_ref, wc_ref, bc_ref,
             wrh_ref, wrab_ref, br_ref, out_ref):
        x = jnp.dot(nbr_ref[...], wh_ref[...], preferred_element_type=jnp.float32)
        x = x + jnp.dot(crd_ref[...], wc_ref[...], preferred_element_type=jnp.float32)
        ch = _ssp(x + bc_ref[...])
        rcol = lax.broadcasted_iota(jnp.int32, (bp, rr_), 1)
        prow = lax.broadcasted_iota(jnp.int32, (bp, rr_), 0) * _NB
        seg = ((rcol >= prow) & (rcol < prow + _NB)).astype(jnp.float32)
        csum = jnp.dot(seg, ch, preferred_element_type=jnp.float32)
        out_ref[...] = (jnp.dot(csum, wrh_ref[...], preferred_element_type=jnp.float32)
                        + jnp.dot(ab_ref[...], wrab_ref[...], preferred_element_type=jnp.float32)
                        + br_ref[...])

    return pl.pallas_call(
        body,
        grid=(_P // bp,),
        in_specs=[
            pl.BlockSpec((rr_, _H), lambda i: (i, 0)),
            pl.BlockSpec((rr_, 8), lambda i: (i, 0)),
            pl.BlockSpec((bp, 8), lambda i: (i, 0)),
            pl.BlockSpec((_H, _CH), lambda i: (0, 0)),
            pl.BlockSpec((8, _CH), lambda i: (0, 0)),
            pl.BlockSpec((1, _CH), lambda i: (0, 0)),
            pl.BlockSpec((_CH, _H), lambda i: (0, 0)),
            pl.BlockSpec((8, _H), lambda i: (0, 0)),
            pl.BlockSpec((1, _H), lambda i: (0, 0)),
        ],
        out_specs=pl.BlockSpec((bp, _H), lambda i: (i, 0)),
        out_shape=jax.ShapeDtypeStruct((_P, _H), jnp.float32),
        interpret=False,
    )(nbr, crd, ab, wh, wc_c, bc2, wrh, wrab, brp)


def kernel(nodes, num_nodes, edges, edges_features, num_edges, node_id_neighbors,
           internal_coordinates_neighbors, num_neighbors, A_dist, B_dist, atom_emb,
           msg_W1, msg_b1, msg_W2, msg_b2, st_W1, st_b1, st_W2, st_b2, Wc, bc, Wr, br):
    f32 = jnp.float32
    n = nodes.reshape(_N)
    off = (jnp.arange(_BATCH, dtype=edges.dtype) * _MAX_NODES)[:, None, None]
    e = (edges + off).reshape(_E, 2)
    src = e[:, 0]
    dst = e[:, 1]
    ef2d = edges_features.reshape(_E, 1).astype(f32)

    w1p = jnp.zeros((_NI, _ESP, _H), f32).at[:, :_ES, :].set(msg_W1)

    npad = jnp.concatenate([n, jnp.zeros((_NP_ - _N,), n.dtype)]).reshape(_NP_, 1)
    embp = jnp.zeros((_H, _H), f32).at[:atom_emb.shape[0]].set(atom_emb)
    h = _emb_call(npad, embp)

    zeros = jnp.zeros((_RPT, _H), f32)
    cab = jnp.asarray(_COLS_A + _COLS_B, dtype=jnp.int32)
    w2ab, b2ab = msg_W2[:, :, cab], msg_b2[:, cab]
    w1p = w1p.astype(jnp.bfloat16)
    w2ab = w2ab.astype(jnp.bfloat16)
    gates = [_gates_call(ef2d, w1p[i], msg_b1[i].reshape(1, _H),
                         w2ab[i], b2ab[i].reshape(1, _H))
             for i in range(_NI)]

    for i in range(_NI):
        p = _sc_interact_call(h, src, dst, gates[i].reshape(_E * (_H // 2)), zeros)
        h = _state_call(p, h, st_W1[i], st_b1[i].reshape(1, _H),
                        st_W2[i], st_b2[i].reshape(1, _H))

    nch_nbr = _P * _NB // _NW // _C
    nidx = node_id_neighbors.reshape(_NW, nch_nbr, _C)
    nbr = _sc_gather_call(h, nidx, nch_nbr)
    crd = jnp.concatenate(
        [internal_coordinates_neighbors.reshape(_P * _NB, 3).astype(f32),
         jnp.zeros((_P * _NB, 5), f32)], axis=1)
    ab = jnp.concatenate(
        [A_dist[:, None].astype(f32), B_dist[:, None].astype(f32),
         jnp.zeros((_P, 6), f32)], axis=1)
    wh = Wc[:_H].astype(f32)
    wc_c = jnp.zeros((8, _CH), f32).at[:3].set(Wc[_H:_H + 3])
    wrh = jnp.zeros((_CH, _H), f32).at[:, :6].set(Wr[:_CH])
    wrab = jnp.zeros((8, _H), f32).at[:2, :6].set(Wr[_CH:_CH + 2])
    brp = jnp.zeros((1, _H), f32).at[0, :6].set(br)
    out = _readout_call(nbr, crd, ab, wh, wc_c, bc.reshape(1, _CH), wrh, wrab, brp)
    return out[:, :6]


# speedup vs baseline: 1.0928x; 1.0008x over previous
import functools
import math

import jax
import jax.numpy as jnp
from jax import lax
from jax.experimental import pallas as pl
from jax.experimental.pallas import tpu as pltpu
from jax.experimental.pallas import tpu_sc as plsc

_BATCH = 4
_MAX_NODES = 2500
_MAX_EDGES = 80000
_P = 1024
_NB = 20
_H = 128
_NI = 3
_STEP = 0.1
_ES = 50
_ESP = 64
_CH = 64
_N = _BATCH * _MAX_NODES
_NP_ = 10240
_E = _BATCH * _MAX_EDGES
_LN2 = math.log(2.0)

_NC = 2
_NS = 16
_NW = _NC * _NS
_C = 80
_ECH = _E // _NW // _C
_RPT = _NP_ // _NS


def _mesh():
    return plsc.VectorSubcoreMesh(
        core_axis_name="c", subcore_axis_name="s",
        num_cores=_NC, num_subcores=_NS)


_COLS_A = [32 * g + j for g in range(_H // 32) for j in range(16)]
_COLS_B = [32 * g + 16 + j for g in range(_H // 32) for j in range(16)]


def _ssp(x):
    return jnp.maximum(x, 0.0) + jnp.log(1.0 + jnp.exp(-jnp.abs(x))) - _LN2


def _sc_gather_call(table, idx, nch):
    total = _NW * nch * _C

    def body(table_hbm, idx_hbm, out_hbm, idx_v, rows_v, sem):
        c = lax.axis_index("c")
        s = lax.axis_index("s")
        wid = s * _NC + c
        pltpu.sync_copy(idx_hbm.at[wid], idx_v)

        def step(j, carry):
            pltpu.async_copy(table_hbm.at[idx_v.at[j]], rows_v, sem).wait()
            pltpu.sync_copy(rows_v, out_hbm.at[pl.ds(wid * nch * _C + j * _C, _C)])
            return carry

        lax.fori_loop(0, nch, step, 0)

    f = pl.kernel(
        body,
        out_type=jax.ShapeDtypeStruct((total, _H), jnp.float32),
        mesh=_mesh(),
        scratch_types=[
            pltpu.VMEM((nch, _C), jnp.int32),
            pltpu.VMEM((_C, _H), jnp.float32),
            pltpu.SemaphoreType.DMA,
        ],
    )
    return f(table, idx)


def _sc_interact_call(h, src, dst, gates, zeros):
    cc_ = 40
    nchk = _E // _NW // cc_
    gw = _H // 2

    def body(h_hbm, src_hbm, dst_hbm, g_hbm, z_hbm, out_hbm,
             src_v, rows_v, gates_v, dst_v,
             lsem0, lsem1, gsem0, gsem1, ssem0, ssem1, agg_sh):
        c = lax.axis_index("c")
        s = lax.axis_index("s")
        wid = s * _NC + c
        lsem = (lsem0, lsem1)
        gsem = (gsem0, gsem1)
        ssem = (ssem0, ssem1)
        pltpu.sync_copy(z_hbm, agg_sh.at[pl.ds(s * _RPT, _RPT)])
        plsc.subcore_barrier()

        ebase = wid * nchk * cc_

        def linear_issue(j, b):
            base = ebase + j * cc_
            pltpu.async_copy(src_hbm.at[pl.ds(base, cc_)], src_v.at[b], lsem[b])
            pltpu.async_copy(dst_hbm.at[pl.ds(base, cc_)], dst_v.at[j % 4], lsem[b])
            pltpu.async_copy(g_hbm.at[pl.ds(base * gw, cc_ * gw)],
                             gates_v.at[pl.ds(b * cc_ * gw, cc_ * gw)], lsem[b])

        def linear_wait(b):
            pltpu.make_async_copy(src_hbm.at[pl.ds(0, cc_)], src_v.at[b], lsem[b]).wait()
            pltpu.make_async_copy(dst_hbm.at[pl.ds(0, cc_)], src_v.at[b], lsem[b]).wait()
            pltpu.make_async_copy(g_hbm.at[pl.ds(0, cc_ * gw)],
                                  gates_v.at[pl.ds(b * cc_ * gw, cc_ * gw)], lsem[b]).wait()

        def gather_issue(b):
            pltpu.async_copy(h_hbm.at[src_v.at[b]], rows_v.at[b], gsem[b])

        def gather_wait(b):
            pltpu.make_async_copy(h_hbm.at[src_v.at[b]], rows_v.at[b], gsem[b]).wait()

        def scatter_issue(j, b):
            pltpu.async_copy(rows_v.at[b], agg_sh.at[dst_v.at[j % 4]], ssem[b], add=True)

        def scatter_wait(b):
            pltpu.make_async_copy(rows_v.at[b], agg_sh.at[pl.ds(0, cc_)], ssem[b]).wait()

        linear_issue(0, 0)
        linear_issue(1, 1)
        linear_wait(0)
        gather_issue(0)

        def gbody(g, carry):
            for b in (0, 1):
                j = 2 * g + b
                nb = 1 - b
                nxt = j + 1 < nchk if b else True

                @pl.when((j >= 1) & nxt)
                def _():
                    scatter_wait(nb)

                @pl.when(nxt)
                def _():
                    linear_wait(nb)
                    gather_issue(nb)

                gather_wait(b)

                @plsc.parallel_loop(0, cc_, 1, unroll=4)
                def mrow(r):
                    for kk in range(_H // 32):
                        goff = pl.multiple_of((b * cc_ + r) * gw + kk * 16, 16)
                        vg = gates_v[pl.ds(goff, 16)]
                        glo = lax.bitcast_convert_type(vg << 16, jnp.float32)
                        ghi = lax.bitcast_convert_type(
                            vg & jnp.int32(-65536), jnp.float32)
                        slo = pl.ds(kk * 32, 16)
                        shi = pl.ds(kk * 32 + 16, 16)
                        rows_v[b, r, slo] = rows_v[b, r, slo] * glo
                        rows_v[b, r, shi] = rows_v[b, r, shi] * ghi

                scatter_issue(j, b)

                @pl.when(j + 2 < nchk)
                def _():
                    linear_issue(j + 2, b)
            return carry

        lax.fori_loop(0, nchk // 2, gbody, 0)
        scatter_wait(0)
        scatter_wait(1)
        plsc.subcore_barrier()
        for k in range(_RPT // cc_):
            off = s * _RPT + k * cc_
            pltpu.sync_copy(agg_sh.at[pl.ds(off, cc_)], rows_v.at[0])
            pltpu.sync_copy(rows_v.at[0], out_hbm.at[c, pl.ds(off, cc_)])

    f = pl.kernel(
        body,
        out_type=jax.ShapeDtypeStruct((2, _NP_, _H), jnp.float32),
        mesh=_mesh(),
        scratch_types=[
            pltpu.VMEM((2, cc_), jnp.int32),
            pltpu.VMEM((2, cc_, _H), jnp.float32),
            pltpu.VMEM((2 * cc_ * _H // 2,), jnp.int32),
            pltpu.VMEM((4, cc_), jnp.int32),
            pltpu.SemaphoreType.DMA,
            pltpu.SemaphoreType.DMA,
            pltpu.SemaphoreType.DMA,
            pltpu.SemaphoreType.DMA,
            pltpu.SemaphoreType.DMA,
            pltpu.SemaphoreType.DMA,
            pltpu.VMEM_SHARED((_NP_, _H), jnp.float32),
        ],
    )
    return f(h, src, dst, gates, zeros)


def _gates_call(ef2d, w1p, b1, w2ab, b2ab):
    be = 640

    def to_bf16_bits(x):
        u = lax.bitcast_convert_type(x, jnp.uint32)
        return (u + 0x7FFF + ((u >> 16) & 1)) >> 16

    def body(ef_ref, w1_ref, b1_ref, w2_ref, b2_ref, g_ref):
        ef = ef_ref[...]
        ki = lax.broadcasted_iota(jnp.int32, (1, _ESP), 1)
        mu = ki.astype(jnp.float32) * _STEP
        msk = ki < _ES
        es = jnp.exp(-((ef - mu) ** 2) * (1.0 / (2.0 * _STEP * _STEP)))
        es = jnp.where(msk, es, 0.0)
        x = jnp.dot(es.astype(jnp.bfloat16), w1_ref[...],
                    preferred_element_type=jnp.float32)
        x = _ssp(x + b1_ref[...])
        g = jnp.dot(x.astype(jnp.bfloat16), w2_ref[...],
                    preferred_element_type=jnp.float32) + b2_ref[...]
        w = to_bf16_bits(g[:, : _H // 2]) | (to_bf16_bits(g[:, _H // 2:]) << 16)
        g_ref[...] = lax.bitcast_convert_type(w, jnp.int32)

    return pl.pallas_call(
        body,
        grid=(_E // be,),
        in_specs=[
            pl.BlockSpec((be, 1), lambda i: (i, 0)),
            pl.BlockSpec((_ESP, _H), lambda i: (0, 0)),
            pl.BlockSpec((1, _H), lambda i: (0, 0)),
            pl.BlockSpec((_H, _H), lambda i: (0, 0)),
            pl.BlockSpec((1, _H), lambda i: (0, 0)),
        ],
        out_specs=pl.BlockSpec((be, _H // 2), lambda i: (i, 0)),
        out_shape=jax.ShapeDtypeStruct((_E, _H // 2), jnp.int32),
        interpret=False,
    )(ef2d, w1p, b1, w2ab, b2ab)


def _emb_call(n2d, emb_pad):
    bn = 512

    def body(n_ref, emb_ref, out_ref):
        oh = (n_ref[...] == lax.broadcasted_iota(jnp.int32, (1, _H), 1))
        out_ref[...] = jnp.dot(oh.astype(jnp.float32), emb_ref[...],
                               preferred_element_type=jnp.float32)

    return pl.pallas_call(
        body,
        grid=(_NP_ // bn,),
        in_specs=[
            pl.BlockSpec((bn, 1), lambda i: (i, 0)),
            pl.BlockSpec((_H, _H), lambda i: (0, 0)),
        ],
        out_specs=pl.BlockSpec((bn, _H), lambda i: (i, 0)),
        out_shape=jax.ShapeDtypeStruct((_NP_, _H), jnp.float32),
        interpret=False,
    )(n2d, emb_pad)


def _state_call(p, h, w1, b1, w2, b2):
    bn = 512

    def body(p0_ref, p1_ref, h_ref, w1_ref, b1_ref, w2_ref, b2_ref, out_ref):
        agg = p0_ref[...] + p1_ref[...]
        x = _ssp(jnp.dot(agg, w1_ref[...], preferred_element_type=jnp.float32) + b1_ref[...])
        out_ref[...] = h_ref[...] + jnp.dot(x, w2_ref[...], preferred_element_type=jnp.float32) + b2_ref[...]

    return pl.pallas_call(
        body,
        grid=(_NP_ // bn,),
        in_specs=[
            pl.BlockSpec((bn, _H), lambda i: (i, 0)),
            pl.BlockSpec((bn, _H), lambda i: (i, 0)),
            pl.BlockSpec((bn, _H), lambda i: (i, 0)),
            pl.BlockSpec((_H, _H), lambda i: (0, 0)),
            pl.BlockSpec((1, _H), lambda i: (0, 0)),
            pl.BlockSpec((_H, _H), lambda i: (0, 0)),
            pl.BlockSpec((1, _H), lambda i: (0, 0)),
        ],
        out_specs=pl.BlockSpec((bn, _H), lambda i: (i, 0)),
        out_shape=jax.ShapeDtypeStruct((_NP_, _H), jnp.float32),
        interpret=False,
    )(p[0], p[1], h, w1, b1, w2, b2)


def _readout_call(nbr, crd, ab, wh, wc_c, bc2, wrh, wrab, brp):
    bp = 128
    rr_ = bp * _NB

    def body(nbr_ref, crd_ref, ab_ref, wh_ref, wc_ref, bc_ref,
             wrh_ref, wrab_ref, br_ref, out_ref):
        x = jnp.dot(nbr_ref[...], wh_ref[...], preferred_element_type=jnp.float32)
        x = x + jnp.dot(crd_ref[...], wc_ref[...], preferred_element_type=jnp.float32)
        ch = _ssp(x + bc_ref[...])
        rcol = lax.broadcasted_iota(jnp.int32, (bp, rr_), 1)
        prow = lax.broadcasted_iota(jnp.int32, (bp, rr_), 0) * _NB
        seg = ((rcol >= prow) & (rcol < prow + _NB)).astype(jnp.float32)
        csum = jnp.dot(seg, ch, preferred_element_type=jnp.float32)
        out_ref[...] = (jnp.dot(csum, wrh_ref[...], preferred_element_type=jnp.float32)
                        + jnp.dot(ab_ref[...], wrab_ref[...], preferred_element_type=jnp.float32)
                        + br_ref[...])

    return pl.pallas_call(
        body,
        grid=(_P // bp,),
        in_specs=[
            pl.BlockSpec((rr_, _H), lambda i: (i, 0)),
            pl.BlockSpec((rr_, 8), lambda i: (i, 0)),
            pl.BlockSpec((bp, 8), lambda i: (i, 0)),
            pl.BlockSpec((_H, _CH), lambda i: (0, 0)),
            pl.BlockSpec((8, _CH), lambda i: (0, 0)),
            pl.BlockSpec((1, _CH), lambda i: (0, 0)),
            pl.BlockSpec((_CH, _H), lambda i: (0, 0)),
            pl.BlockSpec((8, _H), lambda i: (0, 0)),
            pl.BlockSpec((1, _H), lambda i: (0, 0)),
        ],
        out_specs=pl.BlockSpec((bp, _H), lambda i: (i, 0)),
        out_shape=jax.ShapeDtypeStruct((_P, _H), jnp.float32),
        interpret=False,
    )(nbr, crd, ab, wh, wc_c, bc2, wrh, wrab, brp)


def kernel(nodes, num_nodes, edges, edges_features, num_edges, node_id_neighbors,
           internal_coordinates_neighbors, num_neighbors, A_dist, B_dist, atom_emb,
           msg_W1, msg_b1, msg_W2, msg_b2, st_W1, st_b1, st_W2, st_b2, Wc, bc, Wr, br):
    f32 = jnp.float32
    n = nodes.reshape(_N)
    off = (jnp.arange(_BATCH, dtype=edges.dtype) * _MAX_NODES)[:, None, None]
    e = (edges + off).reshape(_E, 2)
    src = e[:, 0]
    dst = e[:, 1]
    ef2d = edges_features.reshape(_E, 1).astype(f32)

    w1p = jnp.zeros((_NI, _ESP, _H), f32).at[:, :_ES, :].set(msg_W1)

    npad = jnp.concatenate([n, jnp.zeros((_NP_ - _N,), n.dtype)]).reshape(_NP_, 1)
    embp = jnp.zeros((_H, _H), f32).at[:atom_emb.shape[0]].set(atom_emb)
    h = _emb_call(npad, embp)

    zeros = jnp.zeros((_RPT, _H), f32)
    cab = jnp.asarray(_COLS_A + _COLS_B, dtype=jnp.int32)
    w2ab, b2ab = msg_W2[:, :, cab], msg_b2[:, cab]
    w1p = w1p.astype(jnp.bfloat16)
    w2ab = w2ab.astype(jnp.bfloat16)
    gates0 = _gates_call(ef2d, w1p[0], msg_b1[0].reshape(1, _H),
                         w2ab[0], b2ab[0].reshape(1, _H))
    gi = gates0
    for i in range(_NI):
        p = _sc_interact_call(h, src, dst, gi.reshape(_E * (_H // 2)), zeros)
        if i + 1 < _NI:
            gi = _gates_call(ef2d, w1p[i + 1], msg_b1[i + 1].reshape(1, _H),
                             w2ab[i + 1], b2ab[i + 1].reshape(1, _H))
        h = _state_call(p, h, st_W1[i], st_b1[i].reshape(1, _H),
                        st_W2[i], st_b2[i].reshape(1, _H))

    nch_nbr = _P * _NB // _NW // _C
    nidx = node_id_neighbors.reshape(_NW, nch_nbr, _C)
    nbr = _sc_gather_call(h, nidx, nch_nbr)
    crd = jnp.concatenate(
        [internal_coordinates_neighbors.reshape(_P * _NB, 3).astype(f32),
         jnp.zeros((_P * _NB, 5), f32)], axis=1)
    ab = jnp.concatenate(
        [A_dist[:, None].astype(f32), B_dist[:, None].astype(f32),
         jnp.zeros((_P, 6), f32)], axis=1)
    wh = Wc[:_H].astype(f32)
    wc_c = jnp.zeros((8, _CH), f32).at[:3].set(Wc[_H:_H + 3])
    wrh = jnp.zeros((_CH, _H), f32).at[:, :6].set(Wr[:_CH])
    wrab = jnp.zeros((8, _H), f32).at[:2, :6].set(Wr[_CH:_CH + 2])
    brp = jnp.zeros((1, _H), f32).at[0, :6].set(br)
    out = _readout_call(nbr, crd, ab, wh, wc_c, bc.reshape(1, _CH), wrh, wrab, brp)
    return out[:, :6]


# speedup vs baseline: 1.3184x; 1.2065x over previous
import functools
import math

import jax
import jax.numpy as jnp
from jax import lax
from jax.experimental import pallas as pl
from jax.experimental.pallas import tpu as pltpu
from jax.experimental.pallas import tpu_sc as plsc

_BATCH = 4
_MAX_NODES = 2500
_MAX_EDGES = 80000
_P = 1024
_NB = 20
_H = 128
_NI = 3
_STEP = 0.1
_ES = 50
_ESP = 64
_CH = 64
_N = _BATCH * _MAX_NODES
_NP_ = 10240
_E = _BATCH * _MAX_EDGES
_LN2 = math.log(2.0)

_NC = 2
_NS = 16
_NW = _NC * _NS
_C = 80
_ECH = _E // _NW // _C
_RPT = _NP_ // _NS


def _mesh():
    return plsc.VectorSubcoreMesh(
        core_axis_name="c", subcore_axis_name="s",
        num_cores=_NC, num_subcores=_NS)


_COLS_A = [32 * g + j for g in range(_H // 32) for j in range(16)]
_COLS_B = [32 * g + 16 + j for g in range(_H // 32) for j in range(16)]


def _ssp(x):
    return jnp.maximum(x, 0.0) + jnp.log(1.0 + jnp.exp(-jnp.abs(x))) - _LN2


def _sc_gather_call(table, idx, nch):
    total = _NW * nch * _C

    def body(table_hbm, idx_hbm, out_hbm, idx_v, rows_v, sem):
        c = lax.axis_index("c")
        s = lax.axis_index("s")
        wid = s * _NC + c
        pltpu.sync_copy(idx_hbm.at[wid], idx_v)

        def step(j, carry):
            pltpu.async_copy(table_hbm.at[idx_v.at[j]], rows_v, sem).wait()
            pltpu.sync_copy(rows_v, out_hbm.at[pl.ds(wid * nch * _C + j * _C, _C)])
            return carry

        lax.fori_loop(0, nch, step, 0)

    f = pl.kernel(
        body,
        out_type=jax.ShapeDtypeStruct((total, _H), jnp.float32),
        mesh=_mesh(),
        scratch_types=[
            pltpu.VMEM((nch, _C), jnp.int32),
            pltpu.VMEM((_C, _H), jnp.float32),
            pltpu.SemaphoreType.DMA,
        ],
    )
    return f(table, idx)


def _sc_interact_call(h, src, dst, gates, zeros):
    cc_ = 40
    nchk = _E // _NW // cc_
    gw = _H // 2

    def body(h_hbm, src_hbm, dst_hbm, g_hbm, z_hbm, out_hbm,
             src_v, rows_v, gates_v, dst_v,
             lsem0, lsem1, gsem0, gsem1, ssem0, ssem1, agg_sh):
        c = lax.axis_index("c")
        s = lax.axis_index("s")
        wid = s * _NC + c
        lsem = (lsem0, lsem1)
        gsem = (gsem0, gsem1)
        ssem = (ssem0, ssem1)
        pltpu.sync_copy(z_hbm, agg_sh.at[pl.ds(s * _RPT, _RPT)])
        plsc.subcore_barrier()

        ebase = wid * nchk * cc_

        def linear_issue(j, b):
            base = ebase + j * cc_
            pltpu.async_copy(src_hbm.at[pl.ds(base, cc_)], src_v.at[b], lsem[b])
            pltpu.async_copy(dst_hbm.at[pl.ds(base, cc_)], dst_v.at[j % 4], lsem[b])
            pltpu.async_copy(g_hbm.at[pl.ds(base, cc_)], gates_v.at[b], lsem[b])

        def linear_wait(b):
            pltpu.make_async_copy(src_hbm.at[pl.ds(0, cc_)], src_v.at[b], lsem[b]).wait()
            pltpu.make_async_copy(dst_hbm.at[pl.ds(0, cc_)], src_v.at[b], lsem[b]).wait()
            pltpu.make_async_copy(g_hbm.at[pl.ds(0, cc_)], gates_v.at[b], lsem[b]).wait()

        def gather_issue(b):
            pltpu.async_copy(h_hbm.at[src_v.at[b]], rows_v.at[b], gsem[b])

        def gather_wait(b):
            pltpu.make_async_copy(h_hbm.at[src_v.at[b]], rows_v.at[b], gsem[b]).wait()

        def scatter_issue(j, b):
            pltpu.async_copy(rows_v.at[b], agg_sh.at[dst_v.at[j % 4]], ssem[b], add=True)

        def scatter_wait(b):
            pltpu.make_async_copy(rows_v.at[b], agg_sh.at[pl.ds(0, cc_)], ssem[b]).wait()

        linear_issue(0, 0)
        linear_issue(1, 1)
        linear_wait(0)
        gather_issue(0)

        def gbody(g, carry):
            for b in (0, 1):
                j = 2 * g + b
                nb = 1 - b
                nxt = j + 1 < nchk if b else True

                @pl.when((j >= 1) & nxt)
                def _():
                    scatter_wait(nb)

                @pl.when(nxt)
                def _():
                    linear_wait(nb)
                    gather_issue(nb)

                gather_wait(b)

                @plsc.parallel_loop(0, cc_, 1, unroll=4)
                def mrow(r):
                    for kk in range(_H // 16):
                        sl = pl.ds(kk * 16, 16)
                        rows_v[b, r, sl] = rows_v[b, r, sl] * gates_v[b, r, sl]

                scatter_issue(j, b)

                @pl.when(j + 2 < nchk)
                def _():
                    linear_issue(j + 2, b)
            return carry

        lax.fori_loop(0, nchk // 2, gbody, 0)
        scatter_wait(0)
        scatter_wait(1)
        plsc.subcore_barrier()
        for k in range(_RPT // cc_):
            off = s * _RPT + k * cc_
            pltpu.sync_copy(agg_sh.at[pl.ds(off, cc_)], rows_v.at[0])
            pltpu.sync_copy(rows_v.at[0], out_hbm.at[c, pl.ds(off, cc_)])

    f = pl.kernel(
        body,
        out_type=jax.ShapeDtypeStruct((2, _NP_, _H), jnp.float32),
        mesh=_mesh(),
        scratch_types=[
            pltpu.VMEM((2, cc_), jnp.int32),
            pltpu.VMEM((2, cc_, _H), jnp.float32),
            pltpu.VMEM((2, cc_, _H), jnp.float32),
            pltpu.VMEM((4, cc_), jnp.int32),
            pltpu.SemaphoreType.DMA,
            pltpu.SemaphoreType.DMA,
            pltpu.SemaphoreType.DMA,
            pltpu.SemaphoreType.DMA,
            pltpu.SemaphoreType.DMA,
            pltpu.SemaphoreType.DMA,
            pltpu.VMEM_SHARED((_NP_, _H), jnp.float32),
        ],
    )
    return f(h, src, dst, gates, zeros)


def _gates_call(ef2d, w1p, b1, w2ab, b2ab):
    be = 640

    def to_bf16_bits(x):
        u = lax.bitcast_convert_type(x, jnp.uint32)
        return (u + 0x7FFF + ((u >> 16) & 1)) >> 16

    def body(ef_ref, w1_ref, b1_ref, w2_ref, b2_ref, g_ref):
        ef = ef_ref[...]
        ki = lax.broadcasted_iota(jnp.int32, (1, _ESP), 1)
        mu = ki.astype(jnp.float32) * _STEP
        msk = ki < _ES
        es = jnp.exp(-((ef - mu) ** 2) * (1.0 / (2.0 * _STEP * _STEP)))
        es = jnp.where(msk, es, 0.0)
        x = jnp.dot(es.astype(jnp.bfloat16), w1_ref[...],
                    preferred_element_type=jnp.float32)
        x = _ssp(x + b1_ref[...])
        g_ref[...] = jnp.dot(x.astype(jnp.bfloat16), w2_ref[...],
                             preferred_element_type=jnp.float32) + b2_ref[...]

    return pl.pallas_call(
        body,
        grid=(_E // be,),
        in_specs=[
            pl.BlockSpec((be, 1), lambda i: (i, 0)),
            pl.BlockSpec((_ESP, _H), lambda i: (0, 0)),
            pl.BlockSpec((1, _H), lambda i: (0, 0)),
            pl.BlockSpec((_H, _H), lambda i: (0, 0)),
            pl.BlockSpec((1, _H), lambda i: (0, 0)),
        ],
        out_specs=pl.BlockSpec((be, _H), lambda i: (i, 0)),
        out_shape=jax.ShapeDtypeStruct((_E, _H), jnp.float32),
        interpret=False,
    )(ef2d, w1p, b1, w2ab, b2ab)


def _emb_call(n2d, emb_pad):
    bn = 512

    def body(n_ref, emb_ref, out_ref):
        oh = (n_ref[...] == lax.broadcasted_iota(jnp.int32, (1, _H), 1))
        out_ref[...] = jnp.dot(oh.astype(jnp.float32), emb_ref[...],
                               preferred_element_type=jnp.float32)

    return pl.pallas_call(
        body,
        grid=(_NP_ // bn,),
        in_specs=[
            pl.BlockSpec((bn, 1), lambda i: (i, 0)),
            pl.BlockSpec((_H, _H), lambda i: (0, 0)),
        ],
        out_specs=pl.BlockSpec((bn, _H), lambda i: (i, 0)),
        out_shape=jax.ShapeDtypeStruct((_NP_, _H), jnp.float32),
        interpret=False,
    )(n2d, emb_pad)


def _state_call(p, h, w1, b1, w2, b2):
    bn = 512

    def body(p0_ref, p1_ref, h_ref, w1_ref, b1_ref, w2_ref, b2_ref, out_ref):
        agg = p0_ref[...] + p1_ref[...]
        x = _ssp(jnp.dot(agg, w1_ref[...], preferred_element_type=jnp.float32) + b1_ref[...])
        out_ref[...] = h_ref[...] + jnp.dot(x, w2_ref[...], preferred_element_type=jnp.float32) + b2_ref[...]

    return pl.pallas_call(
        body,
        grid=(_NP_ // bn,),
        in_specs=[
            pl.BlockSpec((bn, _H), lambda i: (i, 0)),
            pl.BlockSpec((bn, _H), lambda i: (i, 0)),
            pl.BlockSpec((bn, _H), lambda i: (i, 0)),
            pl.BlockSpec((_H, _H), lambda i: (0, 0)),
            pl.BlockSpec((1, _H), lambda i: (0, 0)),
            pl.BlockSpec((_H, _H), lambda i: (0, 0)),
            pl.BlockSpec((1, _H), lambda i: (0, 0)),
        ],
        out_specs=pl.BlockSpec((bn, _H), lambda i: (i, 0)),
        out_shape=jax.ShapeDtypeStruct((_NP_, _H), jnp.float32),
        interpret=False,
    )(p[0], p[1], h, w1, b1, w2, b2)


def _readout_call(nbr, crd, ab, wh, wc_c, bc2, wrh, wrab, brp):
    bp = 128
    rr_ = bp * _NB

    def body(nbr_ref, crd_ref, ab_ref, wh_ref, wc_ref, bc_ref,
             wrh_ref, wrab_ref, br_ref, out_ref):
        x = jnp.dot(nbr_ref[...], wh_ref[...], preferred_element_type=jnp.float32)
        x = x + jnp.dot(crd_ref[...], wc_ref[...], preferred_element_type=jnp.float32)
        ch = _ssp(x + bc_ref[...])
        rcol = lax.broadcasted_iota(jnp.int32, (bp, rr_), 1)
        prow = lax.broadcasted_iota(jnp.int32, (bp, rr_), 0) * _NB
        seg = ((rcol >= prow) & (rcol < prow + _NB)).astype(jnp.float32)
        csum = jnp.dot(seg, ch, preferred_element_type=jnp.float32)
        out_ref[...] = (jnp.dot(csum, wrh_ref[...], preferred_element_type=jnp.float32)
                        + jnp.dot(ab_ref[...], wrab_ref[...], preferred_element_type=jnp.float32)
                        + br_ref[...])

    return pl.pallas_call(
        body,
        grid=(_P // bp,),
        in_specs=[
            pl.BlockSpec((rr_, _H), lambda i: (i, 0)),
            pl.BlockSpec((rr_, 8), lambda i: (i, 0)),
            pl.BlockSpec((bp, 8), lambda i: (i, 0)),
            pl.BlockSpec((_H, _CH), lambda i: (0, 0)),
            pl.BlockSpec((8, _CH), lambda i: (0, 0)),
            pl.BlockSpec((1, _CH), lambda i: (0, 0)),
            pl.BlockSpec((_CH, _H), lambda i: (0, 0)),
            pl.BlockSpec((8, _H), lambda i: (0, 0)),
            pl.BlockSpec((1, _H), lambda i: (0, 0)),
        ],
        out_specs=pl.BlockSpec((bp, _H), lambda i: (i, 0)),
        out_shape=jax.ShapeDtypeStruct((_P, _H), jnp.float32),
        interpret=False,
    )(nbr, crd, ab, wh, wc_c, bc2, wrh, wrab, brp)


def kernel(nodes, num_nodes, edges, edges_features, num_edges, node_id_neighbors,
           internal_coordinates_neighbors, num_neighbors, A_dist, B_dist, atom_emb,
           msg_W1, msg_b1, msg_W2, msg_b2, st_W1, st_b1, st_W2, st_b2, Wc, bc, Wr, br):
    f32 = jnp.float32
    n = nodes.reshape(_N)
    off = (jnp.arange(_BATCH, dtype=edges.dtype) * _MAX_NODES)[:, None, None]
    e = (edges + off).reshape(_E, 2)
    src = e[:, 0]
    dst = e[:, 1]
    ef2d = edges_features.reshape(_E, 1).astype(f32)

    w1p = jnp.zeros((_NI, _ESP, _H), f32).at[:, :_ES, :].set(msg_W1)

    npad = jnp.concatenate([n, jnp.zeros((_NP_ - _N,), n.dtype)]).reshape(_NP_, 1)
    embp = jnp.zeros((_H, _H), f32).at[:atom_emb.shape[0]].set(atom_emb)
    h = _emb_call(npad, embp)

    zeros = jnp.zeros((_RPT, _H), f32)
    w2ab, b2ab = msg_W2, msg_b2
    w1p = w1p.astype(jnp.bfloat16)
    w2ab = w2ab.astype(jnp.bfloat16)
    gates0 = _gates_call(ef2d, w1p[0], msg_b1[0].reshape(1, _H),
                         w2ab[0], b2ab[0].reshape(1, _H))
    gi = gates0
    for i in range(_NI):
        p = _sc_interact_call(h, src, dst, gi, zeros)
        if i + 1 < _NI:
            gi = _gates_call(ef2d, w1p[i + 1], msg_b1[i + 1].reshape(1, _H),
                             w2ab[i + 1], b2ab[i + 1].reshape(1, _H))
        h = _state_call(p, h, st_W1[i], st_b1[i].reshape(1, _H),
                        st_W2[i], st_b2[i].reshape(1, _H))

    nch_nbr = _P * _NB // _NW // _C
    nidx = node_id_neighbors.reshape(_NW, nch_nbr, _C)
    nbr = _sc_gather_call(h, nidx, nch_nbr)
    crd = jnp.concatenate(
        [internal_coordinates_neighbors.reshape(_P * _NB, 3).astype(f32),
         jnp.zeros((_P * _NB, 5), f32)], axis=1)
    ab = jnp.concatenate(
        [A_dist[:, None].astype(f32), B_dist[:, None].astype(f32),
         jnp.zeros((_P, 6), f32)], axis=1)
    wh = Wc[:_H].astype(f32)
    wc_c = jnp.zeros((8, _CH), f32).at[:3].set(Wc[_H:_H + 3])
    wrh = jnp.zeros((_CH, _H), f32).at[:, :6].set(Wr[:_CH])
    wrab = jnp.zeros((8, _H), f32).at[:2, :6].set(Wr[_CH:_CH + 2])
    brp = jnp.zeros((1, _H), f32).at[0, :6].set(br)
    out = _readout_call(nbr, crd, ab, wh, wc_c, bc.reshape(1, _CH), wrh, wrab, brp)
    return out[:, :6]
